# Initial kernel scaffold; baseline (speedup 1.0000x reference)
import jax
import jax.numpy as jnp
from jax.experimental import pallas as pl


def kernel(inputIDs, sequenceIDs, word_emb, pos_emb, seq_emb, gamma, beta):
    raise NotImplementedError("write your pallas kernel here")


# speedup vs baseline: 3.1387x; 3.1387x over previous
import functools

import jax
import jax.numpy as jnp
from jax import lax
from jax.experimental import pallas as pl
from jax.experimental.pallas import tpu as pltpu
from jax.experimental.pallas import tpu_sc as plsc

B, S, H = 1024, 200, 128
VOCAB = 100000
EPS = 1e-12

NC, NS = 2, 16
NW = NC * NS
SEQ_PER_W = B // NW
CA, CB = 128, S - 128
L = 16
HV = H // L


_GATHER_DNUMS = lax.GatherDimensionNumbers(
    offset_dims=(), collapsed_slice_dims=(0,), start_index_map=(0,))


def _shuffle(x, perm):
    return lax.gather(x, perm[:, None], _GATHER_DNUMS, (1,),
                      mode=lax.GatherScatterMode.PROMISE_IN_BOUNDS)


def _lane_sum(x, perms):
    for p in perms:
        x = x + _shuffle(x, p)
    return x


def _ln_token(t, rows_ref, ce_ref, gam, bet, perms):
    xs = []
    for h in range(HV):
        sl = pl.ds(h * L, L)
        xs.append(rows_ref[t, sl] + ce_ref[t, sl])
    s1 = xs[0]
    s2 = xs[0] * xs[0]
    for h in range(1, HV):
        s1 = s1 + xs[h]
        s2 = s2 + xs[h] * xs[h]
    inv_h = jnp.float32(1.0 / H)
    m = _lane_sum(s1, perms) * inv_h
    q = _lane_sum(s2, perms) * inv_h
    v = q - m * m + jnp.float32(EPS)
    i = plsc.bitcast(v, jnp.int32)
    i = jnp.int32(0x5F3759DF) - (i >> 1)
    y = plsc.bitcast(i, jnp.float32)
    half, three_half = jnp.float32(0.5), jnp.float32(1.5)
    for _ in range(3):
        y = y * (three_half - half * v * y * y)
    for h in range(HV):
        sl = pl.ds(h * L, L)
        rows_ref[t, sl] = (xs[h] - m) * (y * gam[h]) + bet[h]


def _sc_body(inp_hbm, sid_hbm, word_hbm, comb_hbm, gamma_hbm, beta_hbm, out_hbm,
             widx_a, widx_b, sid_a, sid_b, cidx_a, cidx_b,
             rows_v, ce_v, gam_v, bet_v, sem):
    wid = lax.axis_index("s") * NC + lax.axis_index("c")

    pltpu.sync_copy(gamma_hbm, gam_v)
    pltpu.sync_copy(beta_hbm, bet_v)
    gam = [gam_v[pl.ds(h * L, L)] for h in range(HV)]
    bet = [bet_v[pl.ds(h * L, L)] for h in range(HV)]
    iota = lax.iota(jnp.int32, L)
    perms = [iota ^ k for k in (8, 4, 2, 1)]

    def seq_body(i, carry):
        b = wid * SEQ_PER_W + i
        pltpu.sync_copy(inp_hbm.at[b, pl.ds(0, CA)], widx_a)
        pltpu.sync_copy(inp_hbm.at[b, pl.ds(CA, CB)], widx_b)
        pltpu.sync_copy(sid_hbm.at[b, pl.ds(0, CA)], sid_a)
        pltpu.sync_copy(sid_hbm.at[b, pl.ds(CA, CB)], sid_b)
        for g in range(CA // L):
            sl = pl.ds(g * L, L)
            cidx_a[sl] = sid_a[sl] * S + (g * L + iota)
        for off in (0, 16, 32, 48, CB - L):
            sl = pl.ds(off, L)
            cidx_b[sl] = sid_b[sl] * S + (CA + off + iota)
        cp1 = pltpu.async_copy(word_hbm.at[widx_a], rows_v.at[pl.ds(0, CA)], sem)
        cp2 = pltpu.async_copy(word_hbm.at[widx_b], rows_v.at[pl.ds(CA, CB)], sem)
        cp3 = pltpu.async_copy(comb_hbm.at[cidx_a], ce_v.at[pl.ds(0, CA)], sem)
        cp4 = pltpu.async_copy(comb_hbm.at[cidx_b], ce_v.at[pl.ds(CA, CB)], sem)
        cp1.wait()
        cp2.wait()
        cp3.wait()
        cp4.wait()

        def token_body(t, c):
            _ln_token(t, rows_v, ce_v, gam, bet, perms)
            return c

        lax.fori_loop(0, S, token_body, 0)
        pltpu.sync_copy(rows_v, out_hbm.at[b])
        return carry

    lax.fori_loop(0, SEQ_PER_W, seq_body, 0)


def kernel(inputIDs, sequenceIDs, word_emb, pos_emb, seq_emb, gamma, beta):
    pe = pos_emb[:S]
    comb = jnp.concatenate([pe + seq_emb[0][None, :], pe + seq_emb[1][None, :]],
                           axis=0)
    f = pl.kernel(
        _sc_body,
        out_type=jax.ShapeDtypeStruct((B, S, H), jnp.float32),
        mesh=plsc.VectorSubcoreMesh(core_axis_name="c", subcore_axis_name="s"),
        compiler_params=pltpu.CompilerParams(needs_layout_passes=False),
        scratch_types=[
            pltpu.VMEM((CA,), jnp.int32),
            pltpu.VMEM((CB,), jnp.int32),
            pltpu.VMEM((CA,), jnp.int32),
            pltpu.VMEM((CB,), jnp.int32),
            pltpu.VMEM((CA,), jnp.int32),
            pltpu.VMEM((CB,), jnp.int32),
            pltpu.VMEM((S, H), jnp.float32),
            pltpu.VMEM((S, H), jnp.float32),
            pltpu.VMEM((H,), jnp.float32),
            pltpu.VMEM((H,), jnp.float32),
            pltpu.SemaphoreType.DMA,
        ],
    )
    return f(inputIDs.astype(jnp.int32), sequenceIDs.astype(jnp.int32),
             word_emb, comb, gamma, beta)


# speedup vs baseline: 3.1542x; 1.0049x over previous
import functools

import jax
import jax.numpy as jnp
from jax import lax
from jax.experimental import pallas as pl
from jax.experimental.pallas import tpu as pltpu
from jax.experimental.pallas import tpu_sc as plsc

B, S, H = 1024, 200, 128
VOCAB = 100000
EPS = 1e-12

NC, NS = 2, 16
NW = NC * NS
SEQ_PER_W = B // NW
CA, CB = 128, S - 128
L = 16
HV = H // L

_GATHER_DNUMS = lax.GatherDimensionNumbers(
    offset_dims=(), collapsed_slice_dims=(0,), start_index_map=(0,))


def _shuffle(x, perm):
    return lax.gather(x, perm[:, None], _GATHER_DNUMS, (1,),
                      mode=lax.GatherScatterMode.PROMISE_IN_BOUNDS)


def _lane_sum(x, perms):
    for p in perms:
        x = x + _shuffle(x, p)
    return x


def _ln_token(t, ci, rows_ref, comb_ref, gam, bet, perms):
    xs = []
    for h in range(HV):
        sl = pl.ds(h * L, L)
        xs.append(rows_ref[t, sl] + comb_ref[ci, sl])
    s1 = xs[0]
    s2 = xs[0] * xs[0]
    for h in range(1, HV):
        s1 = s1 + xs[h]
        s2 = s2 + xs[h] * xs[h]
    inv_h = jnp.float32(1.0 / H)
    m = _lane_sum(s1, perms) * inv_h
    q = _lane_sum(s2, perms) * inv_h
    v = q - m * m + jnp.float32(EPS)
    i = plsc.bitcast(v, jnp.int32)
    i = jnp.int32(0x5F3759DF) - (i >> 1)
    y = plsc.bitcast(i, jnp.float32)
    half, three_half = jnp.float32(0.5), jnp.float32(1.5)
    for _ in range(3):
        y = y * (three_half - half * v * y * y)
    for h in range(HV):
        sl = pl.ds(h * L, L)
        rows_ref[t, sl] = (xs[h] - m) * (y * gam[h]) + bet[h]


def _sc_body(inp_hbm, sid_hbm, word_hbm, comb_hbm, gamma_hbm, beta_hbm, out_hbm,
             widx_a, widx_b, sid_a, sid_b, sid_v, comb_v, rows_v,
             gam_v, bet_v, sem):
    wid = lax.axis_index("s") * NC + lax.axis_index("c")

    pltpu.sync_copy(comb_hbm, comb_v)
    pltpu.sync_copy(gamma_hbm, gam_v)
    pltpu.sync_copy(beta_hbm, bet_v)
    gam = [gam_v[pl.ds(h * L, L)] for h in range(HV)]
    bet = [bet_v[pl.ds(h * L, L)] for h in range(HV)]
    iota = lax.iota(jnp.int32, L)
    perms = [iota ^ k for k in (8, 4, 2, 1)]

    def seq_body(i, carry):
        b = wid * SEQ_PER_W + i
        pltpu.sync_copy(inp_hbm.at[b, pl.ds(0, CA)], widx_a)
        pltpu.sync_copy(inp_hbm.at[b, pl.ds(CA, CB)], widx_b)
        pltpu.sync_copy(sid_hbm.at[b, pl.ds(0, CA)], sid_a)
        pltpu.sync_copy(sid_hbm.at[b, pl.ds(CA, CB)], sid_b)
        for k in range(CA // L):
            sl = pl.ds(k * L, L)
            sid_v[sl] = sid_a[sl]
        for off in (0, 16, 32, 48, CB - L):
            sid_v[pl.ds(CA + off, L)] = sid_b[pl.ds(off, L)]
        cp1 = pltpu.async_copy(word_hbm.at[widx_a], rows_v.at[pl.ds(0, CA)], sem)
        cp2 = pltpu.async_copy(word_hbm.at[widx_b], rows_v.at[pl.ds(CA, CB)], sem)
        cp1.wait()
        cp2.wait()

        def block_body(g, c):
            base = g * 8
            sid16 = sid_v[pl.ds(base, L)]
            for j in range(8):
                t = base + j
                ci = sid16[j] * S + t
                _ln_token(t, ci, rows_v, comb_v, gam, bet, perms)
            return c

        lax.fori_loop(0, S // 8, block_body, 0)
        pltpu.sync_copy(rows_v, out_hbm.at[b])
        return carry

    lax.fori_loop(0, SEQ_PER_W, seq_body, 0)


def kernel(inputIDs, sequenceIDs, word_emb, pos_emb, seq_emb, gamma, beta):
    pe = pos_emb[:S]
    comb = jnp.concatenate([pe + seq_emb[0][None, :], pe + seq_emb[1][None, :]],
                           axis=0)
    f = pl.kernel(
        _sc_body,
        out_type=jax.ShapeDtypeStruct((B, S, H), jnp.float32),
        mesh=plsc.VectorSubcoreMesh(core_axis_name="c", subcore_axis_name="s"),
        compiler_params=pltpu.CompilerParams(needs_layout_passes=False),
        scratch_types=[
            pltpu.VMEM((CA,), jnp.int32),
            pltpu.VMEM((CB,), jnp.int32),
            pltpu.VMEM((CA,), jnp.int32),
            pltpu.VMEM((CB,), jnp.int32),
            pltpu.VMEM((S + 8,), jnp.int32),
            pltpu.VMEM((2 * S, H), jnp.float32),
            pltpu.VMEM((S, H), jnp.float32),
            pltpu.VMEM((H,), jnp.float32),
            pltpu.VMEM((H,), jnp.float32),
            pltpu.SemaphoreType.DMA,
        ],
    )
    return f(inputIDs.astype(jnp.int32), sequenceIDs.astype(jnp.int32),
             word_emb, comb, gamma, beta)


# speedup vs baseline: 4.5229x; 1.4339x over previous
import functools

import jax
import jax.numpy as jnp
from jax import lax
from jax.experimental import pallas as pl
from jax.experimental.pallas import tpu as pltpu
from jax.experimental.pallas import tpu_sc as plsc

B, S, H = 1024, 200, 128
VOCAB = 100000
EPS = 1e-12

NC, NS = 2, 16
NW = NC * NS
SEQ_PER_W = B // NW
TA, TB = 128, S - 128
CA, CB = 104, S - 104
SP = S + 8
L = 16
HV = H // L

_GATHER_DNUMS = lax.GatherDimensionNumbers(
    offset_dims=(), collapsed_slice_dims=(0,), start_index_map=(0,))


def _shuffle(x, perm):
    return lax.gather(x, perm[:, None], _GATHER_DNUMS, (1,),
                      mode=lax.GatherScatterMode.PROMISE_IN_BOUNDS)


def _lane_sum(x, perms):
    for p in perms:
        x = x + _shuffle(x, p)
    return x


def _ln_token(t, ci, we_ref, out_ref, comb_ref, gam, bet, perms):
    xs = []
    for h in range(HV):
        sl = pl.ds(h * L, L)
        xs.append(we_ref[t, sl] + comb_ref[ci, sl])
    s1 = xs[0]
    s2 = xs[0] * xs[0]
    for h in range(1, HV):
        s1 = s1 + xs[h]
        s2 = s2 + xs[h] * xs[h]
    inv_h = jnp.float32(1.0 / H)
    m = _lane_sum(s1, perms) * inv_h
    q = _lane_sum(s2, perms) * inv_h
    v = q - m * m + jnp.float32(EPS)
    i = plsc.bitcast(v, jnp.int32)
    i = jnp.int32(0x5F3759DF) - (i >> 1)
    y = plsc.bitcast(i, jnp.float32)
    half, three_half = jnp.float32(0.5), jnp.float32(1.5)
    for _ in range(2):
        y = y * (three_half - half * v * y * y)
    for h in range(HV):
        sl = pl.ds(h * L, L)
        out_ref[t, sl] = (xs[h] - m) * (y * gam[h]) + bet[h]


def _compute_chunk(t0, nblk, we_ref, out_ref, comb_ref, sid_ref, gam, bet,
                   perms):
    def block_body(g, c):
        base = g * 8
        sid16 = sid_ref[pl.ds(base, L)]
        for j in range(8):
            t = base + j
            ci = sid16[j] * S + (t0 + t)
            _ln_token(t, ci, we_ref, out_ref, comb_ref, gam, bet, perms)
        return c

    lax.fori_loop(0, nblk, block_body, 0)


def _sc_body(inp_hbm, sid_hbm, word_hbm, comb_hbm, gamma_hbm, beta_hbm, out_hbm,
             widx_a, widx_b, sid_a, sid_b, widx_v, sid_v, sidb_v, comb_v,
             we0, we1, out0, out1, gam_v, bet_v,
             isem, gsem0, gsem1, osem0, osem1):
    wid = lax.axis_index("s") * NC + lax.axis_index("c")

    pltpu.sync_copy(comb_hbm, comb_v)
    pltpu.sync_copy(gamma_hbm, gam_v)
    pltpu.sync_copy(beta_hbm, bet_v)
    gam = [gam_v[pl.ds(h * L, L)] for h in range(HV)]
    bet = [bet_v[pl.ds(h * L, L)] for h in range(HV)]
    iota = lax.iota(jnp.int32, L)
    perms = [iota ^ k for k in (8, 4, 2, 1)]

    def load_idx(b):
        c1 = pltpu.async_copy(inp_hbm.at[b, pl.ds(0, TA)], widx_a, isem)
        c2 = pltpu.async_copy(inp_hbm.at[b, pl.ds(TA, TB)], widx_b, isem)
        c3 = pltpu.async_copy(sid_hbm.at[b, pl.ds(0, TA)], sid_a, isem)
        c4 = pltpu.async_copy(sid_hbm.at[b, pl.ds(TA, TB)], sid_b, isem)
        c1.wait()
        c2.wait()
        c3.wait()
        c4.wait()
        for k in range(TA // L):
            sl = pl.ds(k * L, L)
            widx_v[sl] = widx_a[sl]
            sid_v[sl] = sid_a[sl]
        for off in (0, 16, 32, 48, TB - L):
            widx_v[pl.ds(TA + off, L)] = widx_b[pl.ds(off, L)]
            sid_v[pl.ds(TA + off, L)] = sid_b[pl.ds(off, L)]

    def gather_a_desc():
        return pltpu.make_async_copy(
            word_hbm.at[widx_v.at[pl.ds(0, CA)]], we0, gsem0)

    def gather_b_desc():
        return pltpu.make_async_copy(
            word_hbm.at[widx_v.at[pl.ds(CA, CB)]], we1, gsem1)

    def out_desc(b, which):
        if which == 0:
            return pltpu.make_async_copy(out0, out_hbm.at[b, pl.ds(0, CA)],
                                         osem0)
        return pltpu.make_async_copy(out1, out_hbm.at[b, pl.ds(CA, CB)], osem1)

    b0 = wid * SEQ_PER_W
    load_idx(b0)
    gather_a_desc().start()

    def seq_body(g, carry):
        b = wid * SEQ_PER_W + g
        gather_b_desc().start()

        @pl.when(g > 0)
        def _():
            out_desc(b, 0).wait()
        gather_a_desc().wait()
        _compute_chunk(0, CA // 8, we0, out0, comb_v, sid_v, gam, bet, perms)
        out_desc(b, 0).start()

        gather_b_desc().wait()
        for k in range(CB // L):
            sidb_v[pl.ds(k * L, L)] = sid_v[pl.ds(CA + k * L, L)]

        @pl.when(g + 1 < SEQ_PER_W)
        def _():
            load_idx(b + 1)
            gather_a_desc().start()

        @pl.when(g > 0)
        def _():
            out_desc(b, 1).wait()
        _compute_chunk(CA, CB // 8, we1, out1, comb_v, sidb_v, gam, bet, perms)
        out_desc(b, 1).start()
        return carry

    lax.fori_loop(0, SEQ_PER_W, seq_body, 0)
    b_last = wid * SEQ_PER_W + SEQ_PER_W - 1
    out_desc(b_last, 0).wait()
    out_desc(b_last, 1).wait()


def kernel(inputIDs, sequenceIDs, word_emb, pos_emb, seq_emb, gamma, beta):
    pe = pos_emb[:S]
    comb = jnp.concatenate([pe + seq_emb[0][None, :], pe + seq_emb[1][None, :]],
                           axis=0)
    f = pl.kernel(
        _sc_body,
        out_type=jax.ShapeDtypeStruct((B, S, H), jnp.float32),
        mesh=plsc.VectorSubcoreMesh(core_axis_name="c", subcore_axis_name="s"),
        compiler_params=pltpu.CompilerParams(needs_layout_passes=False),
        scratch_types=[
            pltpu.VMEM((TA,), jnp.int32),
            pltpu.VMEM((TB,), jnp.int32),
            pltpu.VMEM((TA,), jnp.int32),
            pltpu.VMEM((TB,), jnp.int32),
            pltpu.VMEM((SP,), jnp.int32),
            pltpu.VMEM((SP,), jnp.int32),
            pltpu.VMEM((CB + 8,), jnp.int32),
            pltpu.VMEM((2 * S, H), jnp.float32),
            pltpu.VMEM((CA, H), jnp.float32),
            pltpu.VMEM((CB, H), jnp.float32),
            pltpu.VMEM((CA, H), jnp.float32),
            pltpu.VMEM((CB, H), jnp.float32),
            pltpu.VMEM((H,), jnp.float32),
            pltpu.VMEM((H,), jnp.float32),
            pltpu.SemaphoreType.DMA,
            pltpu.SemaphoreType.DMA,
            pltpu.SemaphoreType.DMA,
            pltpu.SemaphoreType.DMA,
            pltpu.SemaphoreType.DMA,
        ],
    )
    return f(inputIDs.astype(jnp.int32), sequenceIDs.astype(jnp.int32),
             word_emb, comb, gamma, beta)
